# baseline (speedup 1.0000x reference)
import functools
import math

import jax
import jax.numpy as jnp
from jax import lax
from jax.experimental import pallas as pl
from jax.experimental.pallas import tpu as pltpu
from jax.experimental.pallas import tpu_sc as plsc

_NC = 2
_NS = 16
_D = 128
_HALF = 5120
_GARB = _HALF
_ACC_R = _HALF + 8
_CH = 56
_NB = 1
_NI = 4
_PGRAN = _NS * _CH * _NI
_CHK = 64
_INB = 2048

_mesh = plsc.VectorSubcoreMesh(
    core_axis_name="c", subcore_axis_name="s",
    num_cores=_NC, num_subcores=_NS)


def _partition_build(Epad):
    EPT = Epad // _NS
    NCHUNK = EPT // _INB
    assert NCHUNK * _INB == EPT
    CAP = EPT + _CHK + 16
    def lane_cumsum(x):
        iota16 = lax.iota(jnp.int32, 16)
        for s in (1, 2, 4, 8):
            sh = x.at[jnp.maximum(iota16 - s, 0)].get(
                mode="promise_in_bounds")
            x = x + jnp.where(iota16 >= s, sh, 0)
        return x


    @functools.partial(
        pl.kernel,
        out_type=[
            jax.ShapeDtypeStruct((_NC * Epad,), jnp.int32),
            jax.ShapeDtypeStruct((_NC * Epad,), jnp.int32),
            jax.ShapeDtypeStruct((_NC * _NS,), jnp.int32),
        ],
        mesh=_mesh,
        scratch_types=[
            pltpu.VMEM((_INB,), jnp.int32),
            pltpu.VMEM((_INB,), jnp.int32),
            pltpu.VMEM((CAP,), jnp.int32),
            pltpu.VMEM((CAP,), jnp.int32),
            pltpu.VMEM((_CHK,), jnp.int32),
            pltpu.VMEM((_CHK,), jnp.int32),
            pltpu.VMEM((16,), jnp.int32),
            pltpu.VMEM((16,), jnp.int32),
            pltpu.VMEM_SHARED((_NS * 16,), jnp.int32),
        ],
    )
    def body(srcp_hbm, dstp_hbm, esrc_hbm, edst_hbm, rpt_hbm,
             in_s, in_d, kp_s, kp_d, dum_s, dum_d, word, ptr_ref, cnt_sh):
        cid = lax.axis_index("c")
        sid = lax.axis_index("s")
        ones = jnp.ones((16,), jnp.int32)
        zeros16 = jnp.zeros((16,), jnp.int32)
        garb16 = jnp.full((16,), _GARB, jnp.int32)
        for k in range(_CHK // 16):
            dum_s[pl.ds(k * 16, 16)] = zeros16
            dum_d[pl.ds(k * 16, 16)] = garb16

        ebase = sid * EPT

        def scan_for(side_const):
            def chunk_body(c):
                pltpu.sync_copy(srcp_hbm.at[pl.ds(ebase + c * _INB, _INB)],
                                in_s)
                pltpu.sync_copy(dstp_hbm.at[pl.ds(ebase + c * _INB, _INB)],
                                in_d)

                def vec_body(i):
                    iota16 = lax.iota(jnp.int32, 16)
                    ptr = ptr_ref[...]
                    p0 = ptr[0]
                    s16 = in_s[pl.ds(i * 16, 16)]
                    d16 = in_d[pl.ds(i * 16, 16)]
                    if side_const:
                        keep = d16 >= _HALF
                        dl = d16 - _HALF
                    else:
                        keep = d16 < _HALF
                        dl = d16
                    csum = lane_cumsum(jnp.where(keep, ones, zeros16))
                    lo = zeros16
                    for s in (8, 4, 2, 1):
                        cand = jnp.minimum(lo + s, 16)
                        c = csum.at[cand - 1].get(mode="promise_in_bounds")
                        lo = jnp.where(c <= iota16, cand, lo)
                    inv = jnp.minimum(lo, 15)
                    ss = s16.at[inv].get(mode="promise_in_bounds")
                    sd = jnp.where(keep, dl, 0).at[inv].get(
                        mode="promise_in_bounds")
                    kp_s[pl.ds(p0, 16)] = ss
                    kp_d[pl.ds(p0, 16)] = sd
                    ptr_ref[...] = jnp.full((16,), p0 + csum[15], jnp.int32)

                pl.loop(0, _INB // 16)(vec_body)

            pl.loop(0, NCHUNK)(chunk_body)

        ptr_ref[...] = jnp.zeros((16,), jnp.int32)

        @pl.when(cid == 0)
        def _scan0():
            scan_for(0)

        @pl.when(cid == 1)
        def _scan1():
            scan_for(1)

        ptr = ptr_ref[...]

        cnt = ptr[0]
        for k in range(_CHK // 16):
            kp_s[pl.ds(cnt + k * 16, 16)] = zeros16
            kp_d[pl.ds(cnt + k * 16, 16)] = garb16
        cntp = ((cnt + _CHK - 1) // _CHK) * _CHK

        word[...] = jnp.full((16,), cntp, jnp.int32)
        pltpu.sync_copy(word, cnt_sh.at[pl.ds(sid * 16, 16)])
        plsc.subcore_barrier()
        off = jnp.zeros((), jnp.int32)
        tot = jnp.zeros((), jnp.int32)
        for t in range(_NS):
            pltpu.sync_copy(cnt_sh.at[pl.ds(t * 16, 16)], word)
            c_t = word[...][0]
            off = off + jnp.where(t < sid, c_t, 0)
            tot = tot + c_t

        gbase = pl.multiple_of(cid * Epad + off, _CHK)

        def copy_body(k):
            pltpu.sync_copy(kp_s.at[pl.ds(k * _CHK, _CHK)],
                            esrc_hbm.at[pl.ds(gbase + k * _CHK, _CHK)])
            pltpu.sync_copy(kp_d.at[pl.ds(k * _CHK, _CHK)],
                            edst_hbm.at[pl.ds(gbase + k * _CHK, _CHK)])

        pl.loop(0, cntp // _CHK)(copy_body)

        tot = pl.multiple_of(tot, _CHK)
        totp = jnp.maximum(((tot + _PGRAN - 1) // _PGRAN) * _PGRAN, _PGRAN)

        @pl.when(sid == 0)
        def _tail():
            def pad_body(k):
                pltpu.sync_copy(
                    dum_s, esrc_hbm.at[pl.ds(cid * Epad + tot + k * _CHK,
                                             _CHK)])
                pltpu.sync_copy(
                    dum_d, edst_hbm.at[pl.ds(cid * Epad + tot + k * _CHK,
                                             _CHK)])

            pl.loop(0, (totp - tot) // _CHK)(pad_body)
            word[...] = jnp.full((16,), totp // (_CH * _NS), jnp.int32)
            pltpu.sync_copy(word, rpt_hbm.at[pl.ds(cid * _NS, _NS)])

    return body


def _scatter_build(N, Epad):
    VS = 624
    VTAIL = N - VS * _NS
    ZR = _HALF // _NS
    assert ZR % 8 == 0 and VS % 8 == 0 and VTAIL % 8 == 0

    @functools.partial(
        pl.kernel,
        out_type=jax.ShapeDtypeStruct((_NC * _HALF, _D), jnp.float32),
        mesh=_mesh,
        scratch_types=[
            [pltpu.VMEM((_CH,), jnp.int32) for _ in range(_NI)],
            [pltpu.VMEM((_CH,), jnp.int32) for _ in range(_NI)],
            [pltpu.VMEM((_CH, _D), jnp.float32) for _ in range(_NB)],
            pltpu.VMEM((16,), jnp.int32),
            pltpu.VMEM_SHARED((N, _D), jnp.float32),
            pltpu.VMEM_SHARED((_ACC_R, _D), jnp.float32),
            [pltpu.SemaphoreType.DMA for _ in range(_NI)],
            [pltpu.SemaphoreType.DMA for _ in range(_NI)],
            [pltpu.SemaphoreType.DMA for _ in range(_NB)],
            [pltpu.SemaphoreType.DMA for _ in range(_NB)],
        ],
    )
    def body(v_hbm, esrc_hbm, edst_hbm, rpt_hbm, zero_hbm, out_hbm,
             sidx, didx, rows, word, vsh, acc, isem, jsem, gsem, ssem):
        cid = lax.axis_index("c")
        sid = lax.axis_index("s")
        pltpu.sync_copy(v_hbm.at[pl.ds(sid * VS, VS)],
                        vsh.at[pl.ds(sid * VS, VS)])
        pltpu.sync_copy(zero_hbm.at[pl.ds(0, ZR)],
                        acc.at[pl.ds(sid * ZR, ZR)])

        @pl.when(sid == 0)
        def _tails():
            pltpu.sync_copy(v_hbm.at[pl.ds(_NS * VS, VTAIL)],
                            vsh.at[pl.ds(_NS * VS, VTAIL)])
            pltpu.sync_copy(zero_hbm.at[pl.ds(0, _ACC_R - _HALF)],
                            acc.at[pl.ds(_HALF, _ACC_R - _HALF)])

        pltpu.sync_copy(rpt_hbm.at[pl.ds(cid * _NS, _NS)], word)
        RT = word[...][0]
        ebase = pl.multiple_of(cid * Epad + sid * RT * _CH, _CH)
        plsc.subcore_barrier()

        for q in range(_NI):
            pltpu.async_copy(esrc_hbm.at[pl.ds(ebase + q * _CH, _CH)],
                             sidx[q], isem[q])
            pltpu.async_copy(edst_hbm.at[pl.ds(ebase + q * _CH, _CH)],
                             didx[q], jsem[q])
        for p in range(_NB):
            pltpu.make_async_copy(esrc_hbm.at[pl.ds(ebase, _CH)], sidx[p],
                                  isem[p]).wait()
            pltpu.async_copy(vsh.at[sidx[p]], rows[p], gsem[p])

        @pl.loop(0, RT, step=_NI)
        def _block(J):
            for r in range(_NI):
                rr = J + r
                p = r % _NB
                q = r % _NI
                qn = (r + _NB) % _NI
                pltpu.make_async_copy(edst_hbm.at[pl.ds(ebase, _CH)],
                                      didx[q], jsem[q]).wait()
                pltpu.make_async_copy(vsh.at[sidx[q]], rows[p],
                                      gsem[p]).wait()
                sc = pltpu.async_copy(rows[p], acc.at[didx[q]], ssem[p],
                                      add=True)

                @pl.when(rr + _NI < RT)
                def _():
                    pltpu.async_copy(
                        esrc_hbm.at[pl.ds(ebase + (rr + _NI) * _CH, _CH)],
                        sidx[q], isem[q])
                sc.wait()

                @pl.when(rr + _NI < RT)
                def _():
                    pltpu.async_copy(
                        edst_hbm.at[pl.ds(ebase + (rr + _NI) * _CH, _CH)],
                        didx[q], jsem[q])

                @pl.when(rr + _NB < RT)
                def _():
                    pltpu.make_async_copy(esrc_hbm.at[pl.ds(ebase, _CH)],
                                          sidx[qn], isem[qn]).wait()
                    pltpu.async_copy(vsh.at[sidx[qn]], rows[p], gsem[p])

        plsc.subcore_barrier()
        pltpu.sync_copy(acc.at[pl.ds(sid * ZR, ZR)],
                        out_hbm.at[pl.ds(cid * _HALF + sid * ZR, ZR)])

    return body


_RBLK = 2000


def _mm_kernel(x_ref, w_ref, b_ref, o_ref):
    o_ref[...] = jnp.dot(x_ref[...], w_ref[...],
                         preferred_element_type=jnp.float32) + b_ref[...]


def _mm2_kernel(agg_ref, wo_ref, bo_ref, wv_ref, bv_ref, o_ref):
    h = jnp.dot(agg_ref[...], wo_ref[...],
                preferred_element_type=jnp.float32) + bo_ref[...]
    h = jnp.maximum(h, 0.0)
    o_ref[...] = jnp.dot(h, wv_ref[...],
                         preferred_element_type=jnp.float32) + bv_ref[...]


_W = pl.BlockSpec((_D, _D), lambda i: (0, 0))
_B = pl.BlockSpec((1, _D), lambda i: (0, 0))


def _blk(n):
    return pl.BlockSpec((n, _D), lambda i: (i, 0))


def _mm_bias(x, WT, b):
    N = x.shape[0]
    return pl.pallas_call(
        _mm_kernel, grid=(N // _RBLK,),
        in_specs=[_blk(_RBLK), _W, _B], out_specs=_blk(_RBLK),
        out_shape=jax.ShapeDtypeStruct((N, _D), jnp.float32))(x, WT, b)


def _mm2(agg, WoT, bo, WvT, bv):
    N = agg.shape[0]
    return pl.pallas_call(
        _mm2_kernel, grid=(N // _RBLK,),
        in_specs=[_blk(_RBLK), _W, _B, _W, _B], out_specs=_blk(_RBLK),
        out_shape=jax.ShapeDtypeStruct((N, _D), jnp.float32))(
            agg, WoT, bo, WvT, bv)


def kernel(x, edge_index, Wq0, Wk0, Wv0, Wo0, bq0, bk0, bv0, bo0,
           Wq1, Wk1, Wv1, Wo1, bq1, bk1, bv1, bo1,
           Wq2, Wk2, Wv2, Wo2, bq2, bk2, bv2, bo2):
    N, D = x.shape
    E = edge_index.shape[1]
    per = _NS * _INB
    Epad = ((E + per - 1) // per) * per
    pad = Epad - E
    src = edge_index[0]
    dst = edge_index[1]
    if pad:
        src = jnp.concatenate([src, jnp.zeros((pad,), jnp.int32)])
        dst = jnp.concatenate([dst, jnp.full((pad,), N, jnp.int32)])
    zero_rows = jnp.zeros((_HALF // _NS, D), jnp.float32)

    esrc, edst, rpt = _partition_build(Epad)(src, dst)
    scat = _scatter_build(N, Epad)
    b2 = lambda b: b.reshape(1, -1)

    v = _mm_bias(x, Wv0.T, b2(bv0))
    p = scat(v, esrc, edst, rpt, zero_rows)
    v = _mm2(p[:N], Wo0.T, b2(bo0), Wv1.T, b2(bv1))
    p = scat(v, esrc, edst, rpt, zero_rows)
    v = _mm2(p[:N], Wo1.T, b2(bo1), Wv2.T, b2(bv2))
    p = scat(v, esrc, edst, rpt, zero_rows)
    out = _mm_bias(p[:N], Wo2.T, b2(bo2))
    return out


# speedup vs baseline: 1.3273x; 1.3273x over previous
import functools
import math

import jax
import jax.numpy as jnp
from jax import lax
from jax.experimental import pallas as pl
from jax.experimental.pallas import tpu as pltpu
from jax.experimental.pallas import tpu_sc as plsc

_NC = 2
_NS = 16
_D = 128
_HALF = 5120
_GARB = _HALF
_ACC_R = _HALF + 8
_CH = 24
_NB = 2
_NI = 8
_PGRAN = _NS * _CH * _NI
_CHK = 64
_INB = 2048

_mesh = plsc.VectorSubcoreMesh(
    core_axis_name="c", subcore_axis_name="s",
    num_cores=_NC, num_subcores=_NS)


def _partition_build(Epad):
    EPT = Epad // _NS
    NCHUNK = EPT // _INB
    assert NCHUNK * _INB == EPT
    CAP = EPT + _CHK + 16
    def lane_cumsum(x):
        iota16 = lax.iota(jnp.int32, 16)
        for s in (1, 2, 4, 8):
            sh = x.at[jnp.maximum(iota16 - s, 0)].get(
                mode="promise_in_bounds")
            x = x + jnp.where(iota16 >= s, sh, 0)
        return x


    @functools.partial(
        pl.kernel,
        out_type=[
            jax.ShapeDtypeStruct((_NC * Epad,), jnp.int32),
            jax.ShapeDtypeStruct((_NC * Epad,), jnp.int32),
            jax.ShapeDtypeStruct((_NC * _NS,), jnp.int32),
        ],
        mesh=_mesh,
        scratch_types=[
            pltpu.VMEM((_INB,), jnp.int32),
            pltpu.VMEM((_INB,), jnp.int32),
            pltpu.VMEM((CAP,), jnp.int32),
            pltpu.VMEM((CAP,), jnp.int32),
            pltpu.VMEM((_CHK,), jnp.int32),
            pltpu.VMEM((_CHK,), jnp.int32),
            pltpu.VMEM((16,), jnp.int32),
            pltpu.VMEM((16,), jnp.int32),
            pltpu.VMEM_SHARED((_NS * 16,), jnp.int32),
        ],
    )
    def body(srcp_hbm, dstp_hbm, esrc_hbm, edst_hbm, rpt_hbm,
             in_s, in_d, kp_s, kp_d, dum_s, dum_d, word, ptr_ref, cnt_sh):
        cid = lax.axis_index("c")
        sid = lax.axis_index("s")
        ones = jnp.ones((16,), jnp.int32)
        zeros16 = jnp.zeros((16,), jnp.int32)
        garb16 = jnp.full((16,), _GARB, jnp.int32)
        for k in range(_CHK // 16):
            dum_s[pl.ds(k * 16, 16)] = zeros16
            dum_d[pl.ds(k * 16, 16)] = garb16

        ebase = sid * EPT

        def scan_for(side_const):
            def chunk_body(c):
                pltpu.sync_copy(srcp_hbm.at[pl.ds(ebase + c * _INB, _INB)],
                                in_s)
                pltpu.sync_copy(dstp_hbm.at[pl.ds(ebase + c * _INB, _INB)],
                                in_d)

                def vec_body(i):
                    iota16 = lax.iota(jnp.int32, 16)
                    ptr = ptr_ref[...]
                    p0 = ptr[0]
                    s16 = in_s[pl.ds(i * 16, 16)]
                    d16 = in_d[pl.ds(i * 16, 16)]
                    if side_const:
                        keep = d16 >= _HALF
                        dl = d16 - _HALF
                    else:
                        keep = d16 < _HALF
                        dl = d16
                    csum = lane_cumsum(jnp.where(keep, ones, zeros16))
                    lo = zeros16
                    for s in (8, 4, 2, 1):
                        cand = jnp.minimum(lo + s, 16)
                        c = csum.at[cand - 1].get(mode="promise_in_bounds")
                        lo = jnp.where(c <= iota16, cand, lo)
                    inv = jnp.minimum(lo, 15)
                    ss = s16.at[inv].get(mode="promise_in_bounds")
                    sd = jnp.where(keep, dl, 0).at[inv].get(
                        mode="promise_in_bounds")
                    kp_s[pl.ds(p0, 16)] = ss
                    kp_d[pl.ds(p0, 16)] = sd
                    ptr_ref[...] = jnp.full((16,), p0 + csum[15], jnp.int32)

                pl.loop(0, _INB // 16)(vec_body)

            pl.loop(0, NCHUNK)(chunk_body)

        ptr_ref[...] = jnp.zeros((16,), jnp.int32)

        @pl.when(cid == 0)
        def _scan0():
            scan_for(0)

        @pl.when(cid == 1)
        def _scan1():
            scan_for(1)

        ptr = ptr_ref[...]

        cnt = ptr[0]
        for k in range(_CHK // 16):
            kp_s[pl.ds(cnt + k * 16, 16)] = zeros16
            kp_d[pl.ds(cnt + k * 16, 16)] = garb16
        cntp = ((cnt + _CHK - 1) // _CHK) * _CHK

        word[...] = jnp.full((16,), cntp, jnp.int32)
        pltpu.sync_copy(word, cnt_sh.at[pl.ds(sid * 16, 16)])
        plsc.subcore_barrier()
        off = jnp.zeros((), jnp.int32)
        tot = jnp.zeros((), jnp.int32)
        for t in range(_NS):
            pltpu.sync_copy(cnt_sh.at[pl.ds(t * 16, 16)], word)
            c_t = word[...][0]
            off = off + jnp.where(t < sid, c_t, 0)
            tot = tot + c_t

        gbase = pl.multiple_of(cid * Epad + off, _CHK)

        def copy_body(k):
            pltpu.sync_copy(kp_s.at[pl.ds(k * _CHK, _CHK)],
                            esrc_hbm.at[pl.ds(gbase + k * _CHK, _CHK)])
            pltpu.sync_copy(kp_d.at[pl.ds(k * _CHK, _CHK)],
                            edst_hbm.at[pl.ds(gbase + k * _CHK, _CHK)])

        pl.loop(0, cntp // _CHK)(copy_body)

        tot = pl.multiple_of(tot, _CHK)
        totp = jnp.maximum(((tot + _PGRAN - 1) // _PGRAN) * _PGRAN, _PGRAN)

        @pl.when(sid == 0)
        def _tail():
            def pad_body(k):
                pltpu.sync_copy(
                    dum_s, esrc_hbm.at[pl.ds(cid * Epad + tot + k * _CHK,
                                             _CHK)])
                pltpu.sync_copy(
                    dum_d, edst_hbm.at[pl.ds(cid * Epad + tot + k * _CHK,
                                             _CHK)])

            pl.loop(0, (totp - tot) // _CHK)(pad_body)
            word[...] = jnp.full((16,), totp // (_CH * _NS), jnp.int32)
            pltpu.sync_copy(word, rpt_hbm.at[pl.ds(cid * _NS, _NS)])

    return body


def _scatter_build(N, Epad):
    VS = 624
    VTAIL = N - VS * _NS
    ZR = _HALF // _NS
    assert ZR % 8 == 0 and VS % 8 == 0 and VTAIL % 8 == 0

    @functools.partial(
        pl.kernel,
        out_type=jax.ShapeDtypeStruct((_NC * _HALF, _D), jnp.float32),
        mesh=_mesh,
        scratch_types=[
            [pltpu.VMEM((_CH,), jnp.int32) for _ in range(_NI)],
            [pltpu.VMEM((_CH,), jnp.int32) for _ in range(_NI)],
            [pltpu.VMEM((_CH, _D), jnp.float32) for _ in range(_NB)],
            pltpu.VMEM((16,), jnp.int32),
            pltpu.VMEM_SHARED((N, _D), jnp.float32),
            pltpu.VMEM_SHARED((_ACC_R, _D), jnp.float32),
            [pltpu.SemaphoreType.DMA for _ in range(_NI)],
            [pltpu.SemaphoreType.DMA for _ in range(_NI)],
            [pltpu.SemaphoreType.DMA for _ in range(_NB)],
            [pltpu.SemaphoreType.DMA for _ in range(_NB)],
        ],
    )
    def body(v_hbm, esrc_hbm, edst_hbm, rpt_hbm, zero_hbm, out_hbm,
             sidx, didx, rows, word, vsh, acc, isem, jsem, gsem, ssem):
        cid = lax.axis_index("c")
        sid = lax.axis_index("s")
        pltpu.sync_copy(v_hbm.at[pl.ds(sid * VS, VS)],
                        vsh.at[pl.ds(sid * VS, VS)])
        pltpu.sync_copy(zero_hbm.at[pl.ds(0, ZR)],
                        acc.at[pl.ds(sid * ZR, ZR)])

        @pl.when(sid == 0)
        def _tails():
            pltpu.sync_copy(v_hbm.at[pl.ds(_NS * VS, VTAIL)],
                            vsh.at[pl.ds(_NS * VS, VTAIL)])
            pltpu.sync_copy(zero_hbm.at[pl.ds(0, _ACC_R - _HALF)],
                            acc.at[pl.ds(_HALF, _ACC_R - _HALF)])

        pltpu.sync_copy(rpt_hbm.at[pl.ds(cid * _NS, _NS)], word)
        RT = word[...][0]
        ebase = pl.multiple_of(cid * Epad + sid * RT * _CH, _CH)
        plsc.subcore_barrier()

        for q in range(_NI):
            pltpu.async_copy(esrc_hbm.at[pl.ds(ebase + q * _CH, _CH)],
                             sidx[q], isem[q])
            pltpu.async_copy(edst_hbm.at[pl.ds(ebase + q * _CH, _CH)],
                             didx[q], jsem[q])
        for p in range(_NB):
            pltpu.make_async_copy(esrc_hbm.at[pl.ds(ebase, _CH)], sidx[p],
                                  isem[p]).wait()
            pltpu.async_copy(vsh.at[sidx[p]], rows[p], gsem[p])

        @pl.loop(0, RT, step=_NI)
        def _block(J):
            for r in range(_NI):
                rr = J + r
                p = r % _NB
                q = r % _NI
                qn = (r + _NB) % _NI
                pltpu.make_async_copy(edst_hbm.at[pl.ds(ebase, _CH)],
                                      didx[q], jsem[q]).wait()
                pltpu.make_async_copy(vsh.at[sidx[q]], rows[p],
                                      gsem[p]).wait()
                sc = pltpu.async_copy(rows[p], acc.at[didx[q]], ssem[p],
                                      add=True)

                @pl.when(rr + _NI < RT)
                def _():
                    pltpu.async_copy(
                        esrc_hbm.at[pl.ds(ebase + (rr + _NI) * _CH, _CH)],
                        sidx[q], isem[q])
                sc.wait()

                @pl.when(rr + _NI < RT)
                def _():
                    pltpu.async_copy(
                        edst_hbm.at[pl.ds(ebase + (rr + _NI) * _CH, _CH)],
                        didx[q], jsem[q])

                @pl.when(rr + _NB < RT)
                def _():
                    pltpu.make_async_copy(esrc_hbm.at[pl.ds(ebase, _CH)],
                                          sidx[qn], isem[qn]).wait()
                    pltpu.async_copy(vsh.at[sidx[qn]], rows[p], gsem[p])

        plsc.subcore_barrier()
        pltpu.sync_copy(acc.at[pl.ds(sid * ZR, ZR)],
                        out_hbm.at[pl.ds(cid * _HALF + sid * ZR, ZR)])

    return body


_RBLK = 2000


def _mm_kernel(x_ref, w_ref, b_ref, o_ref):
    o_ref[...] = jnp.dot(x_ref[...], w_ref[...],
                         preferred_element_type=jnp.float32) + b_ref[...]


def _mm2_kernel(agg_ref, wo_ref, bo_ref, wv_ref, bv_ref, o_ref):
    h = jnp.dot(agg_ref[...], wo_ref[...],
                preferred_element_type=jnp.float32) + bo_ref[...]
    h = jnp.maximum(h, 0.0)
    o_ref[...] = jnp.dot(h, wv_ref[...],
                         preferred_element_type=jnp.float32) + bv_ref[...]


_W = pl.BlockSpec((_D, _D), lambda i: (0, 0))
_B = pl.BlockSpec((1, _D), lambda i: (0, 0))


def _blk(n):
    return pl.BlockSpec((n, _D), lambda i: (i, 0))


def _mm_bias(x, WT, b):
    N = x.shape[0]
    return pl.pallas_call(
        _mm_kernel, grid=(N // _RBLK,),
        in_specs=[_blk(_RBLK), _W, _B], out_specs=_blk(_RBLK),
        out_shape=jax.ShapeDtypeStruct((N, _D), jnp.float32))(x, WT, b)


def _mm2(agg, WoT, bo, WvT, bv):
    N = agg.shape[0]
    return pl.pallas_call(
        _mm2_kernel, grid=(N // _RBLK,),
        in_specs=[_blk(_RBLK), _W, _B, _W, _B], out_specs=_blk(_RBLK),
        out_shape=jax.ShapeDtypeStruct((N, _D), jnp.float32))(
            agg, WoT, bo, WvT, bv)


def kernel(x, edge_index, Wq0, Wk0, Wv0, Wo0, bq0, bk0, bv0, bo0,
           Wq1, Wk1, Wv1, Wo1, bq1, bk1, bv1, bo1,
           Wq2, Wk2, Wv2, Wo2, bq2, bk2, bv2, bo2):
    N, D = x.shape
    E = edge_index.shape[1]
    per = _NS * _INB
    Epad = ((E + per - 1) // per) * per
    pad = Epad - E
    src = edge_index[0]
    dst = edge_index[1]
    if pad:
        src = jnp.concatenate([src, jnp.zeros((pad,), jnp.int32)])
        dst = jnp.concatenate([dst, jnp.full((pad,), N, jnp.int32)])
    zero_rows = jnp.zeros((_HALF // _NS, D), jnp.float32)

    esrc, edst, rpt = _partition_build(Epad)(src, dst)
    scat = _scatter_build(N, Epad)
    b2 = lambda b: b.reshape(1, -1)

    v = _mm_bias(x, Wv0.T, b2(bv0))
    p = scat(v, esrc, edst, rpt, zero_rows)
    v = _mm2(p[:N], Wo0.T, b2(bo0), Wv1.T, b2(bv1))
    p = scat(v, esrc, edst, rpt, zero_rows)
    v = _mm2(p[:N], Wo1.T, b2(bo1), Wv2.T, b2(bv2))
    p = scat(v, esrc, edst, rpt, zero_rows)
    out = _mm_bias(p[:N], Wo2.T, b2(bo2))
    return out
